# baseline (speedup 1.0000x reference)
import functools

import jax
import jax.numpy as jnp
import numpy as np
from jax import lax
from jax.experimental import pallas as pl
from jax.experimental.pallas import tpu as pltpu
from jax.experimental.pallas import tpu_sc as plsc

N = 10000
E = 320000
HU = 256
L = 64

NC, NS = 2, 16
NW = NC * NS

E_PAD = 327680
PAD_DST = 10008
N_ACC = 10240

NSPLIT = 2
E_SEG = E_PAD // NSPLIT


G_CH = 1024
G_KI = G_CH // 128
G_EPW = E_SEG // NW
G_NCH = G_EPW // G_CH


def _gather_body(t16_hbm, x16_hbm, src2d_hbm, dst2d_hbm, a_hbm, b_hbm,
                 idx_s, idx_d, rows_a, rows_b, gsem, ssem_a, ssem_b):
    c = lax.axis_index("c")
    s = lax.axis_index("s")
    w = s * NC + c
    for i in range(G_NCH):
        p = i % 2
        e0 = w * G_EPW + i * G_CH
        r0 = w * (G_EPW // 128) + i * G_KI
        pltpu.sync_copy(src2d_hbm.at[pl.ds(r0, G_KI)], idx_s)
        pltpu.sync_copy(dst2d_hbm.at[pl.ds(r0, G_KI)], idx_d)
        if i >= 2:
            pltpu.make_async_copy(rows_a[p], a_hbm.at[pl.ds(0, G_CH)],
                                  ssem_a[p]).wait()
            pltpu.make_async_copy(rows_b[p], b_hbm.at[pl.ds(0, G_CH)],
                                  ssem_b[p]).wait()
        cps = []
        for j in range(G_KI):
            cps.append(pltpu.async_copy(
                t16_hbm.at[idx_s.at[j]], rows_b[p].at[pl.ds(j * 128, 128)],
                gsem))
            cps.append(pltpu.async_copy(
                x16_hbm.at[idx_d.at[j]], rows_a[p].at[pl.ds(j * 128, 128)],
                gsem))
        for cp in cps:
            cp.wait()
        pltpu.async_copy(rows_a[p], a_hbm.at[pl.ds(e0, G_CH)], ssem_a[p])
        pltpu.async_copy(rows_b[p], b_hbm.at[pl.ds(e0, G_CH)], ssem_b[p])
    for p in range(2):
        pltpu.make_async_copy(rows_a[p], a_hbm.at[pl.ds(0, G_CH)],
                              ssem_a[p]).wait()
        pltpu.make_async_copy(rows_b[p], b_hbm.at[pl.ds(0, G_CH)],
                              ssem_b[p]).wait()


def _gather_call(t16, x16, src2d, dst2d):
    mesh = plsc.VectorSubcoreMesh(core_axis_name="c", subcore_axis_name="s")
    fn = functools.partial(
        pl.kernel,
        out_type=(jax.ShapeDtypeStruct((E_SEG, 16), jnp.float32),
                  jax.ShapeDtypeStruct((E_SEG, 16), jnp.float32)),
        mesh=mesh,
        compiler_params=pltpu.CompilerParams(use_tc_tiling_on_sc=False),
        scratch_types=(pltpu.VMEM((G_KI, 128), jnp.int32),
                       pltpu.VMEM((G_KI, 128), jnp.int32),
                       [pltpu.VMEM((G_CH, 16), jnp.float32)] * 2,
                       [pltpu.VMEM((G_CH, 16), jnp.float32)] * 2,
                       pltpu.SemaphoreType.DMA,
                       [pltpu.SemaphoreType.DMA] * 2,
                       [pltpu.SemaphoreType.DMA] * 2),
    )(_gather_body)
    return fn(t16, x16, src2d, dst2d)


S_CH = 128
S_EPT = E_SEG // NS
S_NCHUNK = S_EPT // S_CH
S_IB = 40
S_NSB = S_NCHUNK // S_IB
S_RPT = N_ACC // NS


def _scatter_body(m_hbm, dst2d_hbm, init_hbm, out_hbm, acc_sh, msg_v, idx_v,
                  lsem):
    c = lax.axis_index("c")
    s = lax.axis_index("s")
    col0 = c * 128
    ebase = s * S_EPT
    pltpu.sync_copy(init_hbm.at[pl.ds(s * S_RPT, S_RPT), pl.ds(col0, 128)],
                    acc_sh.at[pl.ds(s * S_RPT, S_RPT)])
    plsc.subcore_barrier()

    def load(k, p):
        pltpu.async_copy(m_hbm.at[pl.ds(ebase + k * S_CH, S_CH),
                                  pl.ds(col0, 128)], msg_v[p], lsem[p])

    def consume(k, r, p):
        pltpu.make_async_copy(
            m_hbm.at[pl.ds(0, S_CH), pl.ds(col0, 128)], msg_v[p],
            lsem[p]).wait()
        pltpu.sync_copy(msg_v[p], acc_sh.at[idx_v.at[r]], add=True)

    load(0, 0)
    load(1, 1)
    for sb in range(S_NSB):
        pltpu.sync_copy(dst2d_hbm.at[pl.ds(s * (S_EPT // 128) + sb * S_IB,
                                           S_IB)], idx_v)
        n_it = S_IB // 2 if sb < S_NSB - 1 else S_IB // 2 - 1

        def body(i, carry, sb=sb):
            k = sb * S_IB + 2 * i
            consume(k, 2 * i, 0)
            load(k + 2, 0)
            consume(k + 1, 2 * i + 1, 1)
            load(k + 3, 1)
            return carry

        lax.fori_loop(0, n_it, body, 0)
        if sb == S_NSB - 1:
            consume(S_NCHUNK - 2, S_IB - 2, 0)
            consume(S_NCHUNK - 1, S_IB - 1, 1)
    plsc.subcore_barrier()
    pltpu.sync_copy(acc_sh.at[pl.ds(s * S_RPT, S_RPT)],
                    out_hbm.at[pl.ds(s * S_RPT, S_RPT), pl.ds(col0, 128)])


def _scatter_call(msgs, dst2d, init):
    mesh = plsc.VectorSubcoreMesh(core_axis_name="c", subcore_axis_name="s")
    fn = functools.partial(
        pl.kernel,
        out_type=jax.ShapeDtypeStruct((N_ACC, HU), jnp.float32),
        mesh=mesh,
        compiler_params=pltpu.CompilerParams(use_tc_tiling_on_sc=True),
        scratch_types=(pltpu.VMEM_SHARED((N_ACC, 128), jnp.float32),
                       [pltpu.VMEM((S_CH, 128), jnp.float32)] * 2,
                       pltpu.VMEM((S_IB, 128), jnp.int32),
                       [pltpu.SemaphoreType.DMA] * 2),
    )(_scatter_body)
    return fn(msgs, dst2d, init)


TE = 2048


def _edge_body(a_ref, b_ref, w1a_ref, w1b_ref, w2_ref, w3_ref, p_ref, vec_ref,
               m_ref):
    f32 = jnp.float32
    vp = vec_ref[...]
    p = p_ref[...]

    def dot2(x):
        xb = x.astype(jnp.bfloat16).astype(f32)
        return (jnp.dot(xb, p, preferred_element_type=f32)
                + jnp.dot(x - xb, p, preferred_element_type=f32))

    def grouped_ln_silu(h, g, be):
        m = dot2(h)
        v = dot2(h * h) - m * m
        hn = (h - m) * lax.rsqrt(v + 1e-5) * g + be
        return hn * jax.nn.sigmoid(hn)

    h = (jnp.dot(a_ref[...], w1a_ref[...], preferred_element_type=f32)
         + jnp.dot(b_ref[...], w1b_ref[...], preferred_element_type=f32)
         + vp[0:1])
    h = grouped_ln_silu(h, vp[1:2], vp[2:3])
    h = jnp.dot(h, w2_ref[...], preferred_element_type=f32) + vp[3:4]
    h = grouped_ln_silu(h, vp[4:5], vp[5:6])
    m_ref[...] = jnp.dot(h, w3_ref[...], preferred_element_type=f32) + vp[6:7]


def _edge_call(a, b, w1a, w1b, w2b, w3b, p, vec_e):
    grid = (E_SEG // TE,)
    wspec = lambda shape: pl.BlockSpec(shape, lambda i: (0, 0))
    return pl.pallas_call(
        _edge_body,
        grid=grid,
        in_specs=[
            pl.BlockSpec((TE, 16), lambda i: (i, 0)),
            pl.BlockSpec((TE, 16), lambda i: (i, 0)),
            wspec((16, HU)),
            wspec((16, HU)),
            wspec((HU, HU)),
            wspec((HU, HU)),
            wspec((HU, HU)),
            wspec((8, HU)),
        ],
        out_specs=pl.BlockSpec((TE, HU), lambda i: (i, 0)),
        out_shape=jax.ShapeDtypeStruct((E_SEG, HU), jnp.float32),
    )(a, b, w1a, w1b, w2b, w3b, p, vec_e)


TN = 2000


def _node_body(s_ref, x_ref, fcw1_ref, fcw2_ref, fcw3_ref, ghw1z_ref,
               ghw1x_ref, ghw2_ref, ghw3_ref, vec_ref, ghb3_ref, o_ref):
    f32 = jnp.float32
    vp = vec_ref[...]

    def ln_silu(h, g, be):
        m = jnp.mean(h, axis=-1, keepdims=True)
        v = jnp.mean(h * h, axis=-1, keepdims=True) - m * m
        hn = (h - m) * lax.rsqrt(v + 1e-5) * g + be
        return hn * jax.nn.sigmoid(hn)

    h = jnp.maximum(s_ref[...], 0.0)
    h = jnp.dot(h, fcw1_ref[...], preferred_element_type=f32) + vp[0:1]
    h = ln_silu(h, vp[1:2], vp[2:3])
    h = jnp.dot(h, fcw2_ref[...], preferred_element_type=f32) + vp[3:4]
    h = ln_silu(h, vp[4:5], vp[5:6])
    z = jnp.dot(h, fcw3_ref[...], preferred_element_type=f32) + vp[6:7]
    h = (jnp.dot(z, ghw1z_ref[...], preferred_element_type=f32)
         + jnp.dot(x_ref[...], ghw1x_ref[...], preferred_element_type=f32)
         + vp[7:8])
    h = ln_silu(h, vp[8:9], vp[9:10])
    h = jnp.dot(h, ghw2_ref[...], preferred_element_type=f32) + vp[10:11]
    h = ln_silu(h, vp[11:12], vp[12:13])
    o_ref[...] = jnp.dot(h, ghw3_ref[...], preferred_element_type=f32) + ghb3_ref[...]


def _node_call(s_acc, x16, fcw1, fcw2, fcw3, ghw1z, ghw1x, ghw2, ghw3, vec_n,
               ghb3):
    grid = (N // TN,)
    wspec = lambda shape: pl.BlockSpec(shape, lambda i: (0, 0))
    return pl.pallas_call(
        _node_body,
        grid=grid,
        in_specs=[
            pl.BlockSpec((TN, HU), lambda i: (i, 0)),
            pl.BlockSpec((TN, 16), lambda i: (i, 0)),
            wspec((HU, L)),
            wspec((L, L)),
            wspec((L, L)),
            wspec((L, L)),
            wspec((16, L)),
            wspec((L, L)),
            wspec((L, 4)),
            wspec((16, L)),
            wspec((1, 4)),
        ],
        out_specs=pl.BlockSpec((TN, 4), lambda i: (i, 0)),
        out_shape=jax.ShapeDtypeStruct((N, 4), jnp.float32),
    )(s_acc, x16, fcw1, fcw2, fcw3, ghw1z, ghw1x, ghw2, ghw3, vec_n, ghb3)


def kernel(x, edge_index, edge_attr, edge_W1, edge_b1, edge_g1, edge_be1,
           edge_W2, edge_b2, edge_g2, edge_be2, edge_W3, edge_b3, fc_W1,
           fc_b1, fc_g1, fc_be1, fc_W2, fc_b2, fc_g2, fc_be2, fc_W3, fc_b3,
           gh_W1, gh_b1, gh_g1, gh_be1, gh_W2, gh_b2, gh_g2, gh_be2, gh_W3,
           gh_b3):
    f32 = jnp.float32
    src = edge_index[0].astype(jnp.int32)
    dst = edge_index[1].astype(jnp.int32)
    pad_e = E_PAD - E
    src2d = jnp.concatenate(
        [src, jnp.zeros((pad_e,), jnp.int32)]).reshape(E_PAD // 128, 128)
    dst2d = jnp.concatenate(
        [dst, jnp.full((pad_e,), PAD_DST, jnp.int32)]).reshape(E_PAD // 128, 128)

    zn8 = jnp.zeros((N, 8), f32)
    t16 = jnp.concatenate([x, edge_attr, zn8], axis=1)
    x16 = jnp.concatenate([x, zn8, zn8[:, :6]], axis=1)

    w1s = jnp.concatenate([edge_W1[k] for k in range(4)], axis=1)
    w1a = jnp.zeros((16, HU), f32).at[0:2].set(w1s[0:2])
    w1b = jnp.zeros((16, HU), f32).at[0:8].set(w1s[2:10])
    w2b = jax.scipy.linalg.block_diag(*[edge_W2[k] for k in range(4)])
    w3b = jax.scipy.linalg.block_diag(*[edge_W3[k] for k in range(4)])
    p = jnp.asarray(np.kron(np.eye(4), np.full((64, 64), 1.0 / 64.0)), f32)
    vec_e = jnp.stack([edge_b1.reshape(-1), edge_g1.reshape(-1),
                       edge_be1.reshape(-1), edge_b2.reshape(-1),
                       edge_g2.reshape(-1), edge_be2.reshape(-1),
                       edge_b3.reshape(-1), jnp.zeros((HU,), f32)])

    ghw1z = gh_W1[0:L]
    ghw1x = jnp.zeros((16, L), f32).at[0:2].set(gh_W1[L:L + 2])
    zl = jnp.zeros((L,), f32)
    vec_n = jnp.stack([fc_b1, fc_g1, fc_be1, fc_b2, fc_g2, fc_be2, fc_b3,
                       gh_b1, gh_g1, gh_be1, gh_b2, gh_g2, gh_be2,
                       zl, zl, zl])
    ghb3 = gh_b3.reshape(1, 4)

    rseg = E_SEG // 128
    msgs = []
    for t in range(NSPLIT):
        src2d_t = lax.slice_in_dim(src2d, t * rseg, (t + 1) * rseg)
        dst2d_t = lax.slice_in_dim(dst2d, t * rseg, (t + 1) * rseg)
        a_t, b_t = _gather_call(t16, x16, src2d_t, dst2d_t)
        msgs.append(_edge_call(a_t, b_t, w1a, w1b, w2b, w3b, p, vec_e))
    s_acc = jnp.zeros((N_ACC, HU), f32)
    for t in range(NSPLIT):
        dst2d_t = lax.slice_in_dim(dst2d, t * rseg, (t + 1) * rseg)
        s_acc = _scatter_call(msgs[t], dst2d_t, s_acc)
    return _node_call(s_acc, x16, fc_W1, fc_W2, fc_W3, ghw1z, ghw1x, gh_W2,
                      gh_W3, vec_n, ghb3)


# speedup vs baseline: 1.0468x; 1.0468x over previous
import functools

import jax
import jax.numpy as jnp
import numpy as np
from jax import lax
from jax.experimental import pallas as pl
from jax.experimental.pallas import tpu as pltpu
from jax.experimental.pallas import tpu_sc as plsc

N = 10000
E = 320000
HU = 256
L = 64

NC, NS = 2, 16
NW = NC * NS

E_PAD = 327680
PAD_DST = 10008
N_ACC = 10240

NSPLIT = 4
E_SEG = E_PAD // NSPLIT


G_CH = 512
G_KI = G_CH // 128
G_EPW = E_SEG // NW
G_NCH = G_EPW // G_CH


def _gather_body(t16_hbm, x16_hbm, src2d_hbm, dst2d_hbm, a_hbm, b_hbm,
                 idx_s, idx_d, rows_a, rows_b, gsem, ssem_a, ssem_b):
    c = lax.axis_index("c")
    s = lax.axis_index("s")
    w = s * NC + c
    for i in range(G_NCH):
        p = i % 2
        e0 = w * G_EPW + i * G_CH
        r0 = w * (G_EPW // 128) + i * G_KI
        pltpu.sync_copy(src2d_hbm.at[pl.ds(r0, G_KI)], idx_s)
        pltpu.sync_copy(dst2d_hbm.at[pl.ds(r0, G_KI)], idx_d)
        if i >= 2:
            pltpu.make_async_copy(rows_a[p], a_hbm.at[pl.ds(0, G_CH)],
                                  ssem_a[p]).wait()
            pltpu.make_async_copy(rows_b[p], b_hbm.at[pl.ds(0, G_CH)],
                                  ssem_b[p]).wait()
        cps = []
        for j in range(G_KI):
            cps.append(pltpu.async_copy(
                t16_hbm.at[idx_s.at[j]], rows_b[p].at[pl.ds(j * 128, 128)],
                gsem))
            cps.append(pltpu.async_copy(
                x16_hbm.at[idx_d.at[j]], rows_a[p].at[pl.ds(j * 128, 128)],
                gsem))
        for cp in cps:
            cp.wait()
        pltpu.async_copy(rows_a[p], a_hbm.at[pl.ds(e0, G_CH)], ssem_a[p])
        pltpu.async_copy(rows_b[p], b_hbm.at[pl.ds(e0, G_CH)], ssem_b[p])
    for p in range(2):
        pltpu.make_async_copy(rows_a[p], a_hbm.at[pl.ds(0, G_CH)],
                              ssem_a[p]).wait()
        pltpu.make_async_copy(rows_b[p], b_hbm.at[pl.ds(0, G_CH)],
                              ssem_b[p]).wait()


def _gather_call(t16, x16, src2d, dst2d):
    mesh = plsc.VectorSubcoreMesh(core_axis_name="c", subcore_axis_name="s")
    fn = functools.partial(
        pl.kernel,
        out_type=(jax.ShapeDtypeStruct((E_SEG, 16), jnp.float32),
                  jax.ShapeDtypeStruct((E_SEG, 16), jnp.float32)),
        mesh=mesh,
        compiler_params=pltpu.CompilerParams(use_tc_tiling_on_sc=False),
        scratch_types=(pltpu.VMEM((G_KI, 128), jnp.int32),
                       pltpu.VMEM((G_KI, 128), jnp.int32),
                       [pltpu.VMEM((G_CH, 16), jnp.float32)] * 2,
                       [pltpu.VMEM((G_CH, 16), jnp.float32)] * 2,
                       pltpu.SemaphoreType.DMA,
                       [pltpu.SemaphoreType.DMA] * 2,
                       [pltpu.SemaphoreType.DMA] * 2),
    )(_gather_body)
    return fn(t16, x16, src2d, dst2d)


S_CH = 128
S_EPT = E_SEG // NS
S_NCHUNK = S_EPT // S_CH
S_IB = 40
S_NSB = S_NCHUNK // S_IB
S_RPT = N_ACC // NS


def _scatter_body(m_hbm, dst2d_hbm, init_hbm, out_hbm, acc_sh, msg_v, idx_v,
                  lsem):
    c = lax.axis_index("c")
    s = lax.axis_index("s")
    col0 = c * 128
    ebase = s * S_EPT
    pltpu.sync_copy(init_hbm.at[pl.ds(s * S_RPT, S_RPT), pl.ds(col0, 128)],
                    acc_sh.at[pl.ds(s * S_RPT, S_RPT)])
    plsc.subcore_barrier()

    def load(k, p):
        pltpu.async_copy(m_hbm.at[pl.ds(ebase + k * S_CH, S_CH),
                                  pl.ds(col0, 128)], msg_v[p], lsem[p])

    def consume(k, r, p):
        pltpu.make_async_copy(
            m_hbm.at[pl.ds(0, S_CH), pl.ds(col0, 128)], msg_v[p],
            lsem[p]).wait()
        pltpu.sync_copy(msg_v[p], acc_sh.at[idx_v.at[r]], add=True)

    load(0, 0)
    load(1, 1)
    for sb in range(S_NSB):
        pltpu.sync_copy(dst2d_hbm.at[pl.ds(s * (S_EPT // 128) + sb * S_IB,
                                           S_IB)], idx_v)
        n_it = S_IB // 2 if sb < S_NSB - 1 else S_IB // 2 - 1

        def body(i, carry, sb=sb):
            k = sb * S_IB + 2 * i
            consume(k, 2 * i, 0)
            load(k + 2, 0)
            consume(k + 1, 2 * i + 1, 1)
            load(k + 3, 1)
            return carry

        lax.fori_loop(0, n_it, body, 0)
        if sb == S_NSB - 1:
            consume(S_NCHUNK - 2, S_IB - 2, 0)
            consume(S_NCHUNK - 1, S_IB - 1, 1)
    plsc.subcore_barrier()
    pltpu.sync_copy(acc_sh.at[pl.ds(s * S_RPT, S_RPT)],
                    out_hbm.at[pl.ds(s * S_RPT, S_RPT), pl.ds(col0, 128)])


def _scatter_call(msgs, dst2d, init):
    mesh = plsc.VectorSubcoreMesh(core_axis_name="c", subcore_axis_name="s")
    fn = functools.partial(
        pl.kernel,
        out_type=jax.ShapeDtypeStruct((N_ACC, HU), jnp.float32),
        mesh=mesh,
        compiler_params=pltpu.CompilerParams(use_tc_tiling_on_sc=True),
        scratch_types=(pltpu.VMEM_SHARED((N_ACC, 128), jnp.float32),
                       [pltpu.VMEM((S_CH, 128), jnp.float32)] * 2,
                       pltpu.VMEM((S_IB, 128), jnp.int32),
                       [pltpu.SemaphoreType.DMA] * 2),
    )(_scatter_body)
    return fn(msgs, dst2d, init)


TE = 2048


def _edge_body(a_ref, b_ref, w1a_ref, w1b_ref, w2_ref, w3_ref, p_ref, vec_ref,
               m_ref):
    f32 = jnp.float32
    vp = vec_ref[...]
    p = p_ref[...]

    def dot2(x):
        xb = x.astype(jnp.bfloat16).astype(f32)
        return (jnp.dot(xb, p, preferred_element_type=f32)
                + jnp.dot(x - xb, p, preferred_element_type=f32))

    def grouped_ln_silu(h, g, be):
        m = dot2(h)
        v = dot2(h * h) - m * m
        hn = (h - m) * lax.rsqrt(v + 1e-5) * g + be
        return hn * jax.nn.sigmoid(hn)

    h = (jnp.dot(a_ref[...], w1a_ref[...], preferred_element_type=f32)
         + jnp.dot(b_ref[...], w1b_ref[...], preferred_element_type=f32)
         + vp[0:1])
    h = grouped_ln_silu(h, vp[1:2], vp[2:3])
    h = jnp.dot(h, w2_ref[...], preferred_element_type=f32) + vp[3:4]
    h = grouped_ln_silu(h, vp[4:5], vp[5:6])
    m_ref[...] = jnp.dot(h, w3_ref[...], preferred_element_type=f32) + vp[6:7]


def _edge_call(a, b, w1a, w1b, w2b, w3b, p, vec_e):
    grid = (E_SEG // TE,)
    wspec = lambda shape: pl.BlockSpec(shape, lambda i: (0, 0))
    return pl.pallas_call(
        _edge_body,
        grid=grid,
        in_specs=[
            pl.BlockSpec((TE, 16), lambda i: (i, 0)),
            pl.BlockSpec((TE, 16), lambda i: (i, 0)),
            wspec((16, HU)),
            wspec((16, HU)),
            wspec((HU, HU)),
            wspec((HU, HU)),
            wspec((HU, HU)),
            wspec((8, HU)),
        ],
        out_specs=pl.BlockSpec((TE, HU), lambda i: (i, 0)),
        out_shape=jax.ShapeDtypeStruct((E_SEG, HU), jnp.float32),
    )(a, b, w1a, w1b, w2b, w3b, p, vec_e)


TN = 2000


def _node_body(s_ref, x_ref, fcw1_ref, fcw2_ref, fcw3_ref, ghw1z_ref,
               ghw1x_ref, ghw2_ref, ghw3_ref, vec_ref, ghb3_ref, o_ref):
    f32 = jnp.float32
    vp = vec_ref[...]

    def ln_silu(h, g, be):
        m = jnp.mean(h, axis=-1, keepdims=True)
        v = jnp.mean(h * h, axis=-1, keepdims=True) - m * m
        hn = (h - m) * lax.rsqrt(v + 1e-5) * g + be
        return hn * jax.nn.sigmoid(hn)

    h = jnp.maximum(s_ref[...], 0.0)
    h = jnp.dot(h, fcw1_ref[...], preferred_element_type=f32) + vp[0:1]
    h = ln_silu(h, vp[1:2], vp[2:3])
    h = jnp.dot(h, fcw2_ref[...], preferred_element_type=f32) + vp[3:4]
    h = ln_silu(h, vp[4:5], vp[5:6])
    z = jnp.dot(h, fcw3_ref[...], preferred_element_type=f32) + vp[6:7]
    h = (jnp.dot(z, ghw1z_ref[...], preferred_element_type=f32)
         + jnp.dot(x_ref[...], ghw1x_ref[...], preferred_element_type=f32)
         + vp[7:8])
    h = ln_silu(h, vp[8:9], vp[9:10])
    h = jnp.dot(h, ghw2_ref[...], preferred_element_type=f32) + vp[10:11]
    h = ln_silu(h, vp[11:12], vp[12:13])
    o_ref[...] = jnp.dot(h, ghw3_ref[...], preferred_element_type=f32) + ghb3_ref[...]


def _node_call(s_acc, x16, fcw1, fcw2, fcw3, ghw1z, ghw1x, ghw2, ghw3, vec_n,
               ghb3):
    grid = (N // TN,)
    wspec = lambda shape: pl.BlockSpec(shape, lambda i: (0, 0))
    return pl.pallas_call(
        _node_body,
        grid=grid,
        in_specs=[
            pl.BlockSpec((TN, HU), lambda i: (i, 0)),
            pl.BlockSpec((TN, 16), lambda i: (i, 0)),
            wspec((HU, L)),
            wspec((L, L)),
            wspec((L, L)),
            wspec((L, L)),
            wspec((16, L)),
            wspec((L, L)),
            wspec((L, 4)),
            wspec((16, L)),
            wspec((1, 4)),
        ],
        out_specs=pl.BlockSpec((TN, 4), lambda i: (i, 0)),
        out_shape=jax.ShapeDtypeStruct((N, 4), jnp.float32),
    )(s_acc, x16, fcw1, fcw2, fcw3, ghw1z, ghw1x, ghw2, ghw3, vec_n, ghb3)


def kernel(x, edge_index, edge_attr, edge_W1, edge_b1, edge_g1, edge_be1,
           edge_W2, edge_b2, edge_g2, edge_be2, edge_W3, edge_b3, fc_W1,
           fc_b1, fc_g1, fc_be1, fc_W2, fc_b2, fc_g2, fc_be2, fc_W3, fc_b3,
           gh_W1, gh_b1, gh_g1, gh_be1, gh_W2, gh_b2, gh_g2, gh_be2, gh_W3,
           gh_b3):
    f32 = jnp.float32
    src = edge_index[0].astype(jnp.int32)
    dst = edge_index[1].astype(jnp.int32)
    pad_e = E_PAD - E
    src2d = jnp.concatenate(
        [src, jnp.zeros((pad_e,), jnp.int32)]).reshape(E_PAD // 128, 128)
    dst2d = jnp.concatenate(
        [dst, jnp.full((pad_e,), PAD_DST, jnp.int32)]).reshape(E_PAD // 128, 128)

    zn8 = jnp.zeros((N, 8), f32)
    t16 = jnp.concatenate([x, edge_attr, zn8], axis=1)
    x16 = jnp.concatenate([x, zn8, zn8[:, :6]], axis=1)

    w1s = jnp.concatenate([edge_W1[k] for k in range(4)], axis=1)
    w1a = jnp.zeros((16, HU), f32).at[0:2].set(w1s[0:2])
    w1b = jnp.zeros((16, HU), f32).at[0:8].set(w1s[2:10])
    w2b = jax.scipy.linalg.block_diag(*[edge_W2[k] for k in range(4)])
    w3b = jax.scipy.linalg.block_diag(*[edge_W3[k] for k in range(4)])
    p = jnp.asarray(np.kron(np.eye(4), np.full((64, 64), 1.0 / 64.0)), f32)
    vec_e = jnp.stack([edge_b1.reshape(-1), edge_g1.reshape(-1),
                       edge_be1.reshape(-1), edge_b2.reshape(-1),
                       edge_g2.reshape(-1), edge_be2.reshape(-1),
                       edge_b3.reshape(-1), jnp.zeros((HU,), f32)])

    ghw1z = gh_W1[0:L]
    ghw1x = jnp.zeros((16, L), f32).at[0:2].set(gh_W1[L:L + 2])
    zl = jnp.zeros((L,), f32)
    vec_n = jnp.stack([fc_b1, fc_g1, fc_be1, fc_b2, fc_g2, fc_be2, fc_b3,
                       gh_b1, gh_g1, gh_be1, gh_b2, gh_g2, gh_be2,
                       zl, zl, zl])
    ghb3 = gh_b3.reshape(1, 4)

    rseg = E_SEG // 128
    msgs = []
    for t in range(NSPLIT):
        src2d_t = lax.slice_in_dim(src2d, t * rseg, (t + 1) * rseg)
        dst2d_t = lax.slice_in_dim(dst2d, t * rseg, (t + 1) * rseg)
        a_t, b_t = _gather_call(t16, x16, src2d_t, dst2d_t)
        msgs.append(_edge_call(a_t, b_t, w1a, w1b, w2b, w3b, p, vec_e))
    s_acc = jnp.zeros((N_ACC, HU), f32)
    for t in range(NSPLIT):
        dst2d_t = lax.slice_in_dim(dst2d, t * rseg, (t + 1) * rseg)
        s_acc = _scatter_call(msgs[t], dst2d_t, s_acc)
    return _node_call(s_acc, x16, fc_W1, fc_W2, fc_W3, ghw1z, ghw1x, gh_W2,
                      gh_W3, vec_n, ghb3)


# speedup vs baseline: 1.3932x; 1.3309x over previous
import functools

import jax
import jax.numpy as jnp
import numpy as np
from jax import lax
from jax.experimental import pallas as pl
from jax.experimental.pallas import tpu as pltpu
from jax.experimental.pallas import tpu_sc as plsc

N = 10000
E = 320000
HU = 256
L = 64

NC, NS = 2, 16
NW = NC * NS

E_PAD = 327680
PAD_DST = 10008
N_ACC = 10240

NSPLIT = 4
E_SEG = E_PAD // NSPLIT


G_CH = 512
G_KI = G_CH // 128
G_EPW = E_SEG // NW
G_NCH = G_EPW // G_CH


def _gather_body(t16_hbm, x16_hbm, src2d_hbm, dst2d_hbm, a_hbm, b_hbm,
                 idx_s, idx_d, rows_a, rows_b, gsem, ssem_a, ssem_b):
    c = lax.axis_index("c")
    s = lax.axis_index("s")
    w = s * NC + c
    for i in range(G_NCH):
        p = i % 2
        e0 = w * G_EPW + i * G_CH
        r0 = w * (G_EPW // 128) + i * G_KI
        pltpu.sync_copy(src2d_hbm.at[pl.ds(r0, G_KI)], idx_s)
        pltpu.sync_copy(dst2d_hbm.at[pl.ds(r0, G_KI)], idx_d)
        if i >= 2:
            pltpu.make_async_copy(rows_a[p], a_hbm.at[pl.ds(0, G_CH)],
                                  ssem_a[p]).wait()
            pltpu.make_async_copy(rows_b[p], b_hbm.at[pl.ds(0, G_CH)],
                                  ssem_b[p]).wait()
        cps = []
        for j in range(G_KI):
            cps.append(pltpu.async_copy(
                t16_hbm.at[idx_s.at[j]], rows_b[p].at[pl.ds(j * 128, 128)],
                gsem))
            cps.append(pltpu.async_copy(
                x16_hbm.at[idx_d.at[j]], rows_a[p].at[pl.ds(j * 128, 128)],
                gsem))
        for cp in cps:
            cp.wait()
        pltpu.async_copy(rows_a[p], a_hbm.at[pl.ds(e0, G_CH)], ssem_a[p])
        pltpu.async_copy(rows_b[p], b_hbm.at[pl.ds(e0, G_CH)], ssem_b[p])
    for p in range(2):
        pltpu.make_async_copy(rows_a[p], a_hbm.at[pl.ds(0, G_CH)],
                              ssem_a[p]).wait()
        pltpu.make_async_copy(rows_b[p], b_hbm.at[pl.ds(0, G_CH)],
                              ssem_b[p]).wait()


def _gather_call(t16, x16, src2d, dst2d):
    mesh = plsc.VectorSubcoreMesh(core_axis_name="c", subcore_axis_name="s")
    fn = functools.partial(
        pl.kernel,
        out_type=(jax.ShapeDtypeStruct((E_SEG, 16), jnp.float32),
                  jax.ShapeDtypeStruct((E_SEG, 16), jnp.float32)),
        mesh=mesh,
        compiler_params=pltpu.CompilerParams(use_tc_tiling_on_sc=False),
        scratch_types=(pltpu.VMEM((G_KI, 128), jnp.int32),
                       pltpu.VMEM((G_KI, 128), jnp.int32),
                       [pltpu.VMEM((G_CH, 16), jnp.float32)] * 2,
                       [pltpu.VMEM((G_CH, 16), jnp.float32)] * 2,
                       pltpu.SemaphoreType.DMA,
                       [pltpu.SemaphoreType.DMA] * 2,
                       [pltpu.SemaphoreType.DMA] * 2),
    )(_gather_body)
    return fn(t16, x16, src2d, dst2d)


S_CH = 128
S_EPT = E_SEG // NS
S_NCHUNK = S_EPT // S_CH
S_IB = 40
S_NSB = S_NCHUNK // S_IB
S_RPT = N_ACC // NS


def _scatter_body(m_hbm, dst2d_hbm, init_hbm, out_hbm, acc_sh, msg_v, idx_v,
                  lsem):
    c = lax.axis_index("c")
    s = lax.axis_index("s")
    col0 = c * 128
    ebase = s * S_EPT
    pltpu.sync_copy(init_hbm.at[pl.ds(s * S_RPT, S_RPT), pl.ds(col0, 128)],
                    acc_sh.at[pl.ds(s * S_RPT, S_RPT)])
    plsc.subcore_barrier()

    def load(k, p):
        pltpu.async_copy(m_hbm.at[pl.ds(ebase + k * S_CH, S_CH),
                                  pl.ds(col0, 128)], msg_v[p], lsem[p])

    def consume(k, r, p):
        pltpu.make_async_copy(
            m_hbm.at[pl.ds(0, S_CH), pl.ds(col0, 128)], msg_v[p],
            lsem[p]).wait()
        pltpu.sync_copy(msg_v[p], acc_sh.at[idx_v.at[r]], add=True)

    load(0, 0)
    load(1, 1)
    for sb in range(S_NSB):
        pltpu.sync_copy(dst2d_hbm.at[pl.ds(s * (S_EPT // 128) + sb * S_IB,
                                           S_IB)], idx_v)
        n_it = S_IB // 2 if sb < S_NSB - 1 else S_IB // 2 - 1

        def body(i, carry, sb=sb):
            k = sb * S_IB + 2 * i
            consume(k, 2 * i, 0)
            load(k + 2, 0)
            consume(k + 1, 2 * i + 1, 1)
            load(k + 3, 1)
            return carry

        lax.fori_loop(0, n_it, body, 0)
        if sb == S_NSB - 1:
            consume(S_NCHUNK - 2, S_IB - 2, 0)
            consume(S_NCHUNK - 1, S_IB - 1, 1)
    plsc.subcore_barrier()
    pltpu.sync_copy(acc_sh.at[pl.ds(s * S_RPT, S_RPT)],
                    out_hbm.at[pl.ds(s * S_RPT, S_RPT), pl.ds(col0, 128)])


def _scatter_call(msgs, dst2d, init):
    mesh = plsc.VectorSubcoreMesh(core_axis_name="c", subcore_axis_name="s")
    fn = functools.partial(
        pl.kernel,
        out_type=jax.ShapeDtypeStruct((N_ACC, HU), jnp.float32),
        mesh=mesh,
        compiler_params=pltpu.CompilerParams(use_tc_tiling_on_sc=True),
        scratch_types=(pltpu.VMEM_SHARED((N_ACC, 128), jnp.float32),
                       [pltpu.VMEM((S_CH, 128), jnp.float32)] * 2,
                       pltpu.VMEM((S_IB, 128), jnp.int32),
                       [pltpu.SemaphoreType.DMA] * 2),
    )(_scatter_body)
    return fn(msgs, dst2d, init)


TE = 8192


def _edge_body(a_ref, b_ref, wa_ref, wb_ref, w2_ref, w3_ref, p_ref, vec_ref,
               m_ref):
    f32 = jnp.float32
    vp = vec_ref[...]
    p = p_ref[...]
    pa = a_ref[...]
    pb = b_ref[...]

    def dot2(x):
        xb = x.astype(jnp.bfloat16).astype(f32)
        return (jnp.dot(xb, p, preferred_element_type=f32)
                + jnp.dot(x - xb, p, preferred_element_type=f32))

    def grouped_ln_silu(h, g, be):
        m = dot2(h)
        v = dot2(h * h) - m * m
        hn = (h - m) * lax.rsqrt(v + 1e-5) * g + be
        return hn * jax.nn.sigmoid(hn)

    for t in range(8):
        h = (jnp.dot(pa, wa_ref[t], preferred_element_type=f32)
             + jnp.dot(pb, wb_ref[t], preferred_element_type=f32)
             + vp[0:1])
        h = grouped_ln_silu(h, vp[1:2], vp[2:3])
        h = jnp.dot(h, w2_ref[...], preferred_element_type=f32) + vp[3:4]
        h = grouped_ln_silu(h, vp[4:5], vp[5:6])
        m_ref[pl.ds(t * (TE // 8), TE // 8), :] = (
            jnp.dot(h, w3_ref[...], preferred_element_type=f32) + vp[6:7])


def _edge_call(a, b, wa, wb, w2b, w3b, p, vec_e):
    grid = (E_SEG // TE,)
    wspec = lambda shape: pl.BlockSpec(shape, lambda i: (0, 0))
    return pl.pallas_call(
        _edge_body,
        grid=grid,
        in_specs=[
            pl.BlockSpec((TE // 8, 128), lambda i: (i, 0)),
            pl.BlockSpec((TE // 8, 128), lambda i: (i, 0)),
            pl.BlockSpec((8, 128, HU), lambda i: (0, 0, 0)),
            pl.BlockSpec((8, 128, HU), lambda i: (0, 0, 0)),
            wspec((HU, HU)),
            wspec((HU, HU)),
            wspec((HU, HU)),
            wspec((8, HU)),
        ],
        out_specs=pl.BlockSpec((TE, HU), lambda i: (i, 0)),
        out_shape=jax.ShapeDtypeStruct((E_SEG, HU), jnp.float32),
    )(a, b, wa, wb, w2b, w3b, p, vec_e)


TN = 2000


def _node_body(s_ref, x_ref, fcw1_ref, fcw2_ref, fcw3_ref, ghw1z_ref,
               ghw1x_ref, ghw2_ref, ghw3_ref, vec_ref, ghb3_ref, o_ref):
    f32 = jnp.float32
    vp = vec_ref[...]

    def ln_silu(h, g, be):
        m = jnp.mean(h, axis=-1, keepdims=True)
        v = jnp.mean(h * h, axis=-1, keepdims=True) - m * m
        hn = (h - m) * lax.rsqrt(v + 1e-5) * g + be
        return hn * jax.nn.sigmoid(hn)

    h = jnp.maximum(s_ref[...], 0.0)
    h = jnp.dot(h, fcw1_ref[...], preferred_element_type=f32) + vp[0:1]
    h = ln_silu(h, vp[1:2], vp[2:3])
    h = jnp.dot(h, fcw2_ref[...], preferred_element_type=f32) + vp[3:4]
    h = ln_silu(h, vp[4:5], vp[5:6])
    z = jnp.dot(h, fcw3_ref[...], preferred_element_type=f32) + vp[6:7]
    h = (jnp.dot(z, ghw1z_ref[...], preferred_element_type=f32)
         + jnp.dot(x_ref[...], ghw1x_ref[...], preferred_element_type=f32)
         + vp[7:8])
    h = ln_silu(h, vp[8:9], vp[9:10])
    h = jnp.dot(h, ghw2_ref[...], preferred_element_type=f32) + vp[10:11]
    h = ln_silu(h, vp[11:12], vp[12:13])
    o_ref[...] = jnp.dot(h, ghw3_ref[...], preferred_element_type=f32) + ghb3_ref[...]


def _node_call(s_acc, x16, fcw1, fcw2, fcw3, ghw1z, ghw1x, ghw2, ghw3, vec_n,
               ghb3):
    grid = (N // TN,)
    wspec = lambda shape: pl.BlockSpec(shape, lambda i: (0, 0))
    return pl.pallas_call(
        _node_body,
        grid=grid,
        in_specs=[
            pl.BlockSpec((TN, HU), lambda i: (i, 0)),
            pl.BlockSpec((TN, 16), lambda i: (i, 0)),
            wspec((HU, L)),
            wspec((L, L)),
            wspec((L, L)),
            wspec((L, L)),
            wspec((16, L)),
            wspec((L, L)),
            wspec((L, 4)),
            wspec((16, L)),
            wspec((1, 4)),
        ],
        out_specs=pl.BlockSpec((TN, 4), lambda i: (i, 0)),
        out_shape=jax.ShapeDtypeStruct((N, 4), jnp.float32),
    )(s_acc, x16, fcw1, fcw2, fcw3, ghw1z, ghw1x, ghw2, ghw3, vec_n, ghb3)


def kernel(x, edge_index, edge_attr, edge_W1, edge_b1, edge_g1, edge_be1,
           edge_W2, edge_b2, edge_g2, edge_be2, edge_W3, edge_b3, fc_W1,
           fc_b1, fc_g1, fc_be1, fc_W2, fc_b2, fc_g2, fc_be2, fc_W3, fc_b3,
           gh_W1, gh_b1, gh_g1, gh_be1, gh_W2, gh_b2, gh_g2, gh_be2, gh_W3,
           gh_b3):
    f32 = jnp.float32
    src = edge_index[0].astype(jnp.int32)
    dst = edge_index[1].astype(jnp.int32)
    pad_e = E_PAD - E
    src2d = jnp.concatenate(
        [src, jnp.zeros((pad_e,), jnp.int32)]).reshape(E_PAD // 128, 128)
    dst_p = jnp.concatenate([dst, jnp.full((pad_e,), PAD_DST, jnp.int32)])
    dst2d = dst_p.reshape(E_PAD // 128, 128)

    zn8 = jnp.zeros((N, 8), f32)
    t16 = jnp.concatenate([x, edge_attr, zn8], axis=1)
    x16 = jnp.concatenate([x, zn8, zn8[:, :6]], axis=1)

    w1s = jnp.concatenate([edge_W1[k] for k in range(4)], axis=1)
    w1a = jnp.zeros((16, HU), f32).at[0:2].set(w1s[0:2])
    w1b = jnp.zeros((16, HU), f32).at[0:8].set(w1s[2:10])
    wa = jnp.zeros((8, 128, HU), f32)
    wb = jnp.zeros((8, 128, HU), f32)
    for t in range(8):
        wa = wa.at[t, 16 * t:16 * t + 16].set(w1a)
        wb = wb.at[t, 16 * t:16 * t + 16].set(w1b)
    w2b = jax.scipy.linalg.block_diag(*[edge_W2[k] for k in range(4)])
    w3b = jax.scipy.linalg.block_diag(*[edge_W3[k] for k in range(4)])
    p = jnp.asarray(np.kron(np.eye(4), np.full((64, 64), 1.0 / 64.0)), f32)
    vec_e = jnp.stack([edge_b1.reshape(-1), edge_g1.reshape(-1),
                       edge_be1.reshape(-1), edge_b2.reshape(-1),
                       edge_g2.reshape(-1), edge_be2.reshape(-1),
                       edge_b3.reshape(-1), jnp.zeros((HU,), f32)])

    ghw1z = gh_W1[0:L]
    ghw1x = jnp.zeros((16, L), f32).at[0:2].set(gh_W1[L:L + 2])
    zl = jnp.zeros((L,), f32)
    vec_n = jnp.stack([fc_b1, fc_g1, fc_be1, fc_b2, fc_g2, fc_be2, fc_b3,
                       gh_b1, gh_g1, gh_be1, gh_b2, gh_g2, gh_be2,
                       zl, zl, zl])
    ghb3 = gh_b3.reshape(1, 4)

    dst_perm = dst_p.reshape(-1, TE // 8, 8).swapaxes(1, 2).reshape(E_PAD)
    dstp2d = dst_perm.reshape(E_PAD // 128, 128)
    rseg = E_SEG // 128
    msgs = []
    for t in range(NSPLIT):
        src2d_t = lax.slice_in_dim(src2d, t * rseg, (t + 1) * rseg)
        dst2d_t = lax.slice_in_dim(dst2d, t * rseg, (t + 1) * rseg)
        a_t, b_t = _gather_call(t16, x16, src2d_t, dst2d_t)
        msgs.append(_edge_call(a_t.reshape(E_SEG // 8, 128),
                               b_t.reshape(E_SEG // 8, 128),
                               wa, wb, w2b, w3b, p, vec_e))
    s_acc = jnp.zeros((N_ACC, HU), f32)
    for t in range(NSPLIT):
        dstp2d_t = lax.slice_in_dim(dstp2d, t * rseg, (t + 1) * rseg)
        s_acc = _scatter_call(msgs[t], dstp2d_t, s_acc)
    return _node_call(s_acc, x16, fc_W1, fc_W2, fc_W3, ghw1z, ghw1x, gh_W2,
                      gh_W3, vec_n, ghb3)
